# baseline (speedup 1.0000x reference)
import functools

import jax
import jax.numpy as jnp
from jax import lax
from jax.experimental import pallas as pl
from jax.experimental.pallas import tpu as pltpu
from jax.experimental.pallas import tpu_sc as plsc

_NC = 2
_NS = 16
_NW = _NC * _NS


def _make_sc_gather(B, bpw):
  mesh = plsc.VectorSubcoreMesh(core_axis_name="c", subcore_axis_name="s")

  @functools.partial(
      pl.kernel,
      mesh=mesh,
      out_type=[
          jax.ShapeDtypeStruct((B * 8, 128), jnp.float32),
          jax.ShapeDtypeStruct((B * 8, 128), jnp.float32),
      ],
      scratch_types=[
          pltpu.VMEM((2, 128), jnp.int32),
          pltpu.VMEM((bpw * 8, 128), jnp.float32),
          pltpu.VMEM((bpw * 8, 128), jnp.float32),
          pltpu.SemaphoreType.DMA,
      ],
  )
  def gather_k(hid_hbm, cell_hbm, idx_hbm, hout, cout,
               idx_v, hrows, crows, sem):
    wid = lax.axis_index("s") * _NC + lax.axis_index("c")
    base = wid * bpw
    pltpu.sync_copy(idx_hbm.at[wid], idx_v)
    copies = []
    for c in range(2):
      copies.append(pltpu.async_copy(
          hid_hbm.at[idx_v.at[c]], hrows.at[pl.ds(c * 128, 128)], sem))
      copies.append(pltpu.async_copy(
          cell_hbm.at[idx_v.at[c]], crows.at[pl.ds(c * 128, 128)], sem))
    for cp in copies:
      cp.wait()
    pltpu.sync_copy(hrows, hout.at[pl.ds(base * 8, bpw * 8)])
    pltpu.sync_copy(crows, cout.at[pl.ds(base * 8, bpw * 8)])

  return gather_k


def _sig(x):
  return 0.5 * (jnp.tanh(0.5 * x) + 1.0)


def _split_chunks(v):
  l0 = jnp.concatenate([v[:, 0, :], v[:, 2, :], v[:, 4, :], v[:, 6, :]],
                       axis=-1)
  l1 = jnp.concatenate([v[:, 1, :], v[:, 3, :], v[:, 5, :], v[:, 7, :]],
                       axis=-1)
  return l0, l1


def _lstm_body(x_ref, h3_ref, c3_ref,
               wi0_ref, wh0_ref, b0_ref, wi1_ref, wh1_ref, b1_ref, op_ref,
               nh3_ref, nc3_ref, top_ref, *, H):
  dn = (((1,), (1,)), ((), ()))
  prec = None
  h0, h1 = _split_chunks(h3_ref[...])
  c0, c1 = _split_chunks(c3_ref[...])
  x = x_ref[...]
  g0 = lax.dot_general(x, wi0_ref[...], dn, precision=prec,
                       preferred_element_type=jnp.float32)
  g0 = g0 + lax.dot_general(h0, wh0_ref[...], dn, precision=prec,
                            preferred_element_type=jnp.float32)
  g0 = g0 + b0_ref[...]
  i0, f0, gg0, o0 = (g0[:, :H], g0[:, H:2 * H], g0[:, 2 * H:3 * H],
                     g0[:, 3 * H:])
  cn0 = _sig(f0) * c0 + _sig(i0) * jnp.tanh(gg0)
  hn0 = _sig(o0) * jnp.tanh(cn0)
  g1 = lax.dot_general(hn0, wi1_ref[...], dn, precision=prec,
                       preferred_element_type=jnp.float32)
  g1 = g1 + lax.dot_general(h1, wh1_ref[...], dn, precision=prec,
                            preferred_element_type=jnp.float32)
  g1 = g1 + b1_ref[...]
  i1, f1, gg1, o1 = (g1[:, :H], g1[:, H:2 * H], g1[:, 2 * H:3 * H],
                     g1[:, 3 * H:])
  cn1 = _sig(f1) * c1 + _sig(i1) * jnp.tanh(gg1)
  hn1 = _sig(o1) * jnp.tanh(cn1)
  for ht in range(4):
    sl = slice(ht * 128, (ht + 1) * 128)
    nh3_ref[:, 2 * ht, :] = hn0[:, sl]
    nh3_ref[:, 2 * ht + 1, :] = hn1[:, sl]
    nc3_ref[:, 2 * ht, :] = cn0[:, sl]
    nc3_ref[:, 2 * ht + 1, :] = cn1[:, sl]
  top_ref[...] = jnp.where(op_ref[...] == 1, hn1, h1)


def _lstm_call(x, h3, c3, wi0, wh0, b0, wi1, wh1, b1, op2):
  B, IN = x.shape
  H = 512
  G = 4 * H
  bm = 128
  grid = (B // bm,)
  bspec_act = pl.BlockSpec((bm, IN), lambda i: (i, 0))
  bspec_h = pl.BlockSpec((bm, 512), lambda i: (i, 0))
  bspec_3 = pl.BlockSpec((bm, 8, 128), lambda i: (i, 0, 0))
  bspec_w = pl.BlockSpec((G, IN), lambda i: (0, 0))
  bspec_wh = pl.BlockSpec((G, H), lambda i: (0, 0))
  bspec_b = pl.BlockSpec((1, G), lambda i: (0, 0))
  bspec_op = pl.BlockSpec((bm, 1), lambda i: (i, 0))
  out_shape = [jax.ShapeDtypeStruct((B, 8, 128), jnp.float32),
               jax.ShapeDtypeStruct((B, 8, 128), jnp.float32),
               jax.ShapeDtypeStruct((B, H), jnp.float32)]
  out_specs = [bspec_3, bspec_3, bspec_h]
  return pl.pallas_call(
      functools.partial(_lstm_body, H=H),
      grid=grid,
      in_specs=[bspec_act, bspec_3, bspec_3,
                bspec_w, bspec_wh, bspec_b, bspec_wh, bspec_wh, bspec_b,
                bspec_op],
      out_specs=out_specs,
      out_shape=out_shape,
      compiler_params=pltpu.CompilerParams(
          dimension_semantics=("arbitrary",)),
  )(x, h3, c3, wi0, wh0, b0, wi1, wh1, b1, op2)


def kernel(input, hidden_stack, cell_stack, W_ih0, W_hh0, b0,
           W_ih1, W_hh1, b1, pos, op):
  B = input.shape[0]
  S1, _, H, L = hidden_stack.shape
  hid_t = (hidden_stack.reshape(S1, B, 4, 128, 2)
           .transpose(0, 1, 2, 4, 3).reshape(S1 * B * 8, 128))
  cell_t = (cell_stack.reshape(S1, B, 4, 128, 2)
            .transpose(0, 1, 2, 4, 3).reshape(S1 * B * 8, 128))
  pos32 = pos.astype(jnp.int32)
  bpw = B // _NW
  bidx = jnp.arange(B, dtype=jnp.int32)
  rowbase = (pos32 * B + bidx) * 8
  idx = (rowbase[:, None] + jnp.arange(8, dtype=jnp.int32)[None, :]
         ).reshape(_NW, 2, 128)
  gather_k = _make_sc_gather(B, bpw)
  hout, cout = gather_k(hid_t, cell_t, idx)
  h3 = hout.reshape(B, 8, 128)
  c3 = cout.reshape(B, 8, 128)
  op2 = op.astype(jnp.int32).reshape(B, 1)
  nh3, nc3, top = _lstm_call(
      input, h3, c3, W_ih0, W_hh0, b0.reshape(1, 4 * H),
      W_ih1, W_hh1, b1.reshape(1, 4 * H), op2)
  next_hidden = (nh3.reshape(B, 4, 2, 128).transpose(0, 1, 3, 2)
                 .reshape(B, H, L))
  next_cell = (nc3.reshape(B, 4, 2, 128).transpose(0, 1, 3, 2)
               .reshape(B, H, L))
  return (top, next_hidden, next_cell)


# speedup vs baseline: 1.2003x; 1.2003x over previous
import functools

import jax
import jax.numpy as jnp
from jax import lax
from jax.experimental import pallas as pl
from jax.experimental.pallas import tpu as pltpu
from jax.experimental.pallas import tpu_sc as plsc

_NC = 2
_NS = 16
_NW = _NC * _NS


def _make_sc_gather(B, bpw):
  mesh = plsc.VectorSubcoreMesh(core_axis_name="c", subcore_axis_name="s")

  @functools.partial(
      pl.kernel,
      mesh=mesh,
      out_type=[
          jax.ShapeDtypeStruct((B * 8, 128), jnp.float32),
          jax.ShapeDtypeStruct((B * 8, 128), jnp.float32),
      ],
      scratch_types=[
          pltpu.VMEM((bpw,), jnp.int32),
          pltpu.VMEM((2, 128), jnp.int32),
          pltpu.VMEM((bpw * 8, 128), jnp.float32),
          pltpu.VMEM((bpw * 8, 128), jnp.float32),
          pltpu.SemaphoreType.DMA,
          pltpu.SemaphoreType.DMA,
      ],
  )
  def gather_k(hid_hbm, cell_hbm, pos_hbm, hout, cout,
               pos_v, idx_v, hrows, crows, sem, wsem):
    wid = lax.axis_index("s") * _NC + lax.axis_index("c")
    base = wid * bpw
    pltpu.sync_copy(pos_hbm.at[pl.ds(base, bpw)], pos_v)
    lane = lax.iota(jnp.int32, 16)
    dnum = lax.GatherDimensionNumbers(
        offset_dims=(), collapsed_slice_dims=(0,), start_index_map=(0,))
    for c in range(2):
      p16 = pos_v[pl.ds(c * 16, 16)]
      for r in range(8):
        f = r * 16 + lane
        exl = lax.shift_right_logical(f, 3)
        k = lax.bitwise_and(f, 7)
        p = lax.gather(p16, exl[:, None], dnum, slice_sizes=(1,),
                       mode=lax.GatherScatterMode.PROMISE_IN_BOUNDS)
        row = (p * B + base + c * 16 + exl) * 8 + k
        idx_v[c, pl.ds(r * 16, 16)] = row
    hcopies = [pltpu.async_copy(
        hid_hbm.at[idx_v.at[c]], hrows.at[pl.ds(c * 128, 128)], sem)
        for c in range(2)]
    ccopies = [pltpu.async_copy(
        cell_hbm.at[idx_v.at[c]], crows.at[pl.ds(c * 128, 128)], wsem)
        for c in range(2)]
    for cp in hcopies:
      cp.wait()
    wh = pltpu.async_copy(hrows, hout.at[pl.ds(base * 8, bpw * 8)], sem)
    for cp in ccopies:
      cp.wait()
    wc = pltpu.async_copy(crows, cout.at[pl.ds(base * 8, bpw * 8)], wsem)
    wh.wait()
    wc.wait()

  return gather_k


def _sig(x):
  return 0.5 * (jnp.tanh(0.5 * x) + 1.0)


def _split_chunks(v):
  l0 = jnp.concatenate([v[:, 0, :], v[:, 2, :], v[:, 4, :], v[:, 6, :]],
                       axis=-1)
  l1 = jnp.concatenate([v[:, 1, :], v[:, 3, :], v[:, 5, :], v[:, 7, :]],
                       axis=-1)
  return l0, l1


def _lstm_body(x_ref, h3_ref, c3_ref,
               wi0_ref, wh0_ref, b0_ref, wi1_ref, wh1_ref, b1_ref, op_ref,
               nh3_ref, nc3_ref, top_ref, *, H):
  dn = (((1,), (1,)), ((), ()))
  prec = None
  h0, h1 = _split_chunks(h3_ref[...])
  c0, c1 = _split_chunks(c3_ref[...])
  x = x_ref[...]
  g0 = lax.dot_general(x, wi0_ref[...], dn, precision=prec,
                       preferred_element_type=jnp.float32)
  g0 = g0 + lax.dot_general(h0, wh0_ref[...], dn, precision=prec,
                            preferred_element_type=jnp.float32)
  g0 = g0 + b0_ref[...]
  i0, f0, gg0, o0 = (g0[:, :H], g0[:, H:2 * H], g0[:, 2 * H:3 * H],
                     g0[:, 3 * H:])
  cn0 = _sig(f0) * c0 + _sig(i0) * jnp.tanh(gg0)
  hn0 = _sig(o0) * jnp.tanh(cn0)
  g1 = lax.dot_general(hn0, wi1_ref[...], dn, precision=prec,
                       preferred_element_type=jnp.float32)
  g1 = g1 + lax.dot_general(h1, wh1_ref[...], dn, precision=prec,
                            preferred_element_type=jnp.float32)
  g1 = g1 + b1_ref[...]
  i1, f1, gg1, o1 = (g1[:, :H], g1[:, H:2 * H], g1[:, 2 * H:3 * H],
                     g1[:, 3 * H:])
  cn1 = _sig(f1) * c1 + _sig(i1) * jnp.tanh(gg1)
  hn1 = _sig(o1) * jnp.tanh(cn1)
  for ht in range(4):
    sl = slice(ht * 128, (ht + 1) * 128)
    nh3_ref[:, 2 * ht, :] = hn0[:, sl]
    nh3_ref[:, 2 * ht + 1, :] = hn1[:, sl]
    nc3_ref[:, 2 * ht, :] = cn0[:, sl]
    nc3_ref[:, 2 * ht + 1, :] = cn1[:, sl]
  top_ref[...] = jnp.where(op_ref[...] == 1, hn1, h1)


def _lstm_call(x, h3, c3, wi0, wh0, b0, wi1, wh1, b1, op2):
  B, IN = x.shape
  H = 512
  G = 4 * H
  bm = 256
  grid = (B // bm,)
  bspec_act = pl.BlockSpec((bm, IN), lambda i: (i, 0))
  bspec_h = pl.BlockSpec((bm, 512), lambda i: (i, 0))
  bspec_3 = pl.BlockSpec((bm, 8, 128), lambda i: (i, 0, 0))
  bspec_w = pl.BlockSpec((G, IN), lambda i: (0, 0))
  bspec_wh = pl.BlockSpec((G, H), lambda i: (0, 0))
  bspec_b = pl.BlockSpec((1, G), lambda i: (0, 0))
  bspec_op = pl.BlockSpec((bm, 1), lambda i: (i, 0))
  out_shape = [jax.ShapeDtypeStruct((B, 8, 128), jnp.float32),
               jax.ShapeDtypeStruct((B, 8, 128), jnp.float32),
               jax.ShapeDtypeStruct((B, H), jnp.float32)]
  out_specs = [bspec_3, bspec_3, bspec_h]
  return pl.pallas_call(
      functools.partial(_lstm_body, H=H),
      grid=grid,
      in_specs=[bspec_act, bspec_3, bspec_3,
                bspec_w, bspec_wh, bspec_b, bspec_wh, bspec_wh, bspec_b,
                bspec_op],
      out_specs=out_specs,
      out_shape=out_shape,
      compiler_params=pltpu.CompilerParams(
          dimension_semantics=("arbitrary",)),
  )(x, h3, c3, wi0, wh0, b0, wi1, wh1, b1, op2)


def kernel(input, hidden_stack, cell_stack, W_ih0, W_hh0, b0,
           W_ih1, W_hh1, b1, pos, op):
  B = input.shape[0]
  S1, _, H, L = hidden_stack.shape
  hid_t = (hidden_stack.reshape(S1, B, 4, 128, 2)
           .transpose(0, 1, 2, 4, 3).reshape(S1 * B * 8, 128))
  cell_t = (cell_stack.reshape(S1, B, 4, 128, 2)
            .transpose(0, 1, 2, 4, 3).reshape(S1 * B * 8, 128))
  pos32 = pos.astype(jnp.int32)
  bpw = B // _NW
  gather_k = _make_sc_gather(B, bpw)
  hout, cout = gather_k(hid_t, cell_t, pos32)
  h3 = hout.reshape(B, 8, 128)
  c3 = cout.reshape(B, 8, 128)
  op2 = op.astype(jnp.int32).reshape(B, 1)
  nh3, nc3, top = _lstm_call(
      input, h3, c3, W_ih0, W_hh0, b0.reshape(1, 4 * H),
      W_ih1, W_hh1, b1.reshape(1, 4 * H), op2)
  next_hidden = (nh3.reshape(B, 4, 2, 128).transpose(0, 1, 3, 2)
                 .reshape(B, H, L))
  next_cell = (nc3.reshape(B, 4, 2, 128).transpose(0, 1, 3, 2)
               .reshape(B, H, L))
  return (top, next_hidden, next_cell)


# speedup vs baseline: 1.2773x; 1.0642x over previous
import functools

import jax
import jax.numpy as jnp
from jax import lax
from jax.experimental import pallas as pl
from jax.experimental.pallas import tpu as pltpu
from jax.experimental.pallas import tpu_sc as plsc

_NC = 2
_NS = 16
_NW = _NC * _NS


def _make_sc_gather(B, bpw):
  mesh = plsc.VectorSubcoreMesh(core_axis_name="c", subcore_axis_name="s")

  @functools.partial(
      pl.kernel,
      mesh=mesh,
      out_type=[
          jax.ShapeDtypeStruct((B * 4, 128), jnp.float32),
          jax.ShapeDtypeStruct((B * 4, 128), jnp.float32),
          jax.ShapeDtypeStruct((B * 4, 128), jnp.float32),
          jax.ShapeDtypeStruct((B * 4, 128), jnp.float32),
      ],
      scratch_types=[
          pltpu.VMEM((bpw,), jnp.int32),
          pltpu.VMEM((2, 128), jnp.int32),
          pltpu.VMEM((bpw * 8, 128), jnp.float32),
          pltpu.VMEM((bpw * 8, 128), jnp.float32),
          pltpu.SemaphoreType.DMA,
          pltpu.SemaphoreType.DMA,
      ],
  )
  def gather_k(hid_hbm, cell_hbm, pos_hbm, h0out, h1out, c0out, c1out,
               pos_v, idx_v, hrows, crows, sem, wsem):
    wid = lax.axis_index("s") * _NC + lax.axis_index("c")
    base = wid * bpw
    pltpu.sync_copy(pos_hbm.at[pl.ds(base, bpw)], pos_v)
    lane = lax.iota(jnp.int32, 16)
    dnum = lax.GatherDimensionNumbers(
        offset_dims=(), collapsed_slice_dims=(0,), start_index_map=(0,))
    for r in range(16):
      l = r // 8
      btl = (r // 2) & 3
      ht = 2 * (r % 2) + lax.shift_right_logical(lane, 3)
      bi = lax.bitwise_and(lane, 7)
      p16 = pos_v[pl.ds((btl // 2) * 16, 16)]
      gi = (btl & 1) * 8 + bi
      p = lax.gather(p16, gi[:, None], dnum, slice_sizes=(1,),
                     mode=lax.GatherScatterMode.PROMISE_IN_BOUNDS)
      b = base + btl * 8 + bi
      row = (p * B + b) * 8 + ht * 2 + l
      idx_v[r // 8, pl.ds((r % 8) * 16, 16)] = row
    hcopies = [pltpu.async_copy(
        hid_hbm.at[idx_v.at[c]], hrows.at[pl.ds(c * 128, 128)], sem)
        for c in range(2)]
    ccopies = [pltpu.async_copy(
        cell_hbm.at[idx_v.at[c]], crows.at[pl.ds(c * 128, 128)], wsem)
        for c in range(2)]
    dst = pl.ds(wid * 128, 128)
    for cp in hcopies:
      cp.wait()
    w0 = pltpu.async_copy(hrows.at[pl.ds(0, 128)], h0out.at[dst], sem)
    w1 = pltpu.async_copy(hrows.at[pl.ds(128, 128)], h1out.at[dst], sem)
    for cp in ccopies:
      cp.wait()
    w2 = pltpu.async_copy(crows.at[pl.ds(0, 128)], c0out.at[dst], wsem)
    w3 = pltpu.async_copy(crows.at[pl.ds(128, 128)], c1out.at[dst], wsem)
    for wcp in (w0, w1, w2, w3):
      wcp.wait()

  return gather_k


def _sig(x):
  return 0.5 * (jnp.tanh(0.5 * x) + 1.0)


def _lstm_body(x_ref, h0_ref, h1_ref, c0_ref, c1_ref,
               wi0_ref, wh0_ref, b0_ref, wi1_ref, wh1_ref, b1_ref, op_ref,
               nh3_ref, nc3_ref, top_ref, *, H):
  dn = (((1,), (1,)), ((), ()))
  prec = None
  h0, h1 = h0_ref[...], h1_ref[...]
  c0, c1 = c0_ref[...], c1_ref[...]
  x = x_ref[...]
  g0 = lax.dot_general(x, wi0_ref[...], dn, precision=prec,
                       preferred_element_type=jnp.float32)
  g0 = g0 + lax.dot_general(h0, wh0_ref[...], dn, precision=prec,
                            preferred_element_type=jnp.float32)
  g0 = g0 + b0_ref[...]
  i0, f0, gg0, o0 = (g0[:, :H], g0[:, H:2 * H], g0[:, 2 * H:3 * H],
                     g0[:, 3 * H:])
  cn0 = _sig(f0) * c0 + _sig(i0) * jnp.tanh(gg0)
  hn0 = _sig(o0) * jnp.tanh(cn0)
  g1 = lax.dot_general(hn0, wi1_ref[...], dn, precision=prec,
                       preferred_element_type=jnp.float32)
  g1 = g1 + lax.dot_general(h1, wh1_ref[...], dn, precision=prec,
                            preferred_element_type=jnp.float32)
  g1 = g1 + b1_ref[...]
  i1, f1, gg1, o1 = (g1[:, :H], g1[:, H:2 * H], g1[:, 2 * H:3 * H],
                     g1[:, 3 * H:])
  cn1 = _sig(f1) * c1 + _sig(i1) * jnp.tanh(gg1)
  hn1 = _sig(o1) * jnp.tanh(cn1)
  for ht in range(4):
    sl = slice(ht * 128, (ht + 1) * 128)
    nh3_ref[:, 2 * ht, :] = hn0[:, sl]
    nh3_ref[:, 2 * ht + 1, :] = hn1[:, sl]
    nc3_ref[:, 2 * ht, :] = cn0[:, sl]
    nc3_ref[:, 2 * ht + 1, :] = cn1[:, sl]
  top_ref[...] = jnp.where(op_ref[...] == 1, hn1, h1)


def _lstm_call(x, h0, h1, c0, c1, wi0, wh0, b0, wi1, wh1, b1, op2):
  B, IN = x.shape
  H = 512
  G = 4 * H
  bm = 256
  grid = (B // bm,)
  bspec_act = pl.BlockSpec((bm, IN), lambda i: (i, 0))
  bspec_h = pl.BlockSpec((bm, 512), lambda i: (i, 0))
  bspec_3 = pl.BlockSpec((bm, 8, 128), lambda i: (i, 0, 0))
  bspec_w = pl.BlockSpec((G, IN), lambda i: (0, 0))
  bspec_wh = pl.BlockSpec((G, H), lambda i: (0, 0))
  bspec_b = pl.BlockSpec((1, G), lambda i: (0, 0))
  bspec_op = pl.BlockSpec((bm, 1), lambda i: (i, 0))
  out_shape = [jax.ShapeDtypeStruct((B, 8, 128), jnp.float32),
               jax.ShapeDtypeStruct((B, 8, 128), jnp.float32),
               jax.ShapeDtypeStruct((B, H), jnp.float32)]
  out_specs = [bspec_3, bspec_3, bspec_h]
  return pl.pallas_call(
      functools.partial(_lstm_body, H=H),
      grid=grid,
      in_specs=[bspec_act, bspec_h, bspec_h, bspec_h, bspec_h,
                bspec_w, bspec_wh, bspec_b, bspec_wh, bspec_wh, bspec_b,
                bspec_op],
      out_specs=out_specs,
      out_shape=out_shape,
      compiler_params=pltpu.CompilerParams(
          dimension_semantics=("arbitrary",)),
  )(x, h0, h1, c0, c1, wi0, wh0, b0, wi1, wh1, b1, op2)


def kernel(input, hidden_stack, cell_stack, W_ih0, W_hh0, b0,
           W_ih1, W_hh1, b1, pos, op):
  B = input.shape[0]
  S1, _, H, L = hidden_stack.shape
  hid_t = (hidden_stack.reshape(S1, B, 4, 128, 2)
           .transpose(0, 1, 2, 4, 3).reshape(S1 * B * 8, 128))
  cell_t = (cell_stack.reshape(S1, B, 4, 128, 2)
            .transpose(0, 1, 2, 4, 3).reshape(S1 * B * 8, 128))
  pos32 = pos.astype(jnp.int32)
  bpw = B // _NW
  gather_k = _make_sc_gather(B, bpw)
  h0r, h1r, c0r, c1r = gather_k(hid_t, cell_t, pos32)

  def _as_bh(rows):
    return (rows.reshape(B // 8, 4, 8, 128).transpose(0, 2, 1, 3)
            .reshape(B, 512))

  h0, h1, c0, c1 = map(_as_bh, (h0r, h1r, c0r, c1r))
  op2 = op.astype(jnp.int32).reshape(B, 1)
  nh3, nc3, top = _lstm_call(
      input, h0, h1, c0, c1, W_ih0, W_hh0, b0.reshape(1, 4 * H),
      W_ih1, W_hh1, b1.reshape(1, 4 * H), op2)
  next_hidden = (nh3.reshape(B, 4, 2, 128).transpose(0, 1, 3, 2)
                 .reshape(B, H, L))
  next_cell = (nc3.reshape(B, 4, 2, 128).transpose(0, 1, 3, 2)
               .reshape(B, H, L))
  return (top, next_hidden, next_cell)


# speedup vs baseline: 1.2851x; 1.0061x over previous
import functools

import jax
import jax.numpy as jnp
from jax import lax
from jax.experimental import pallas as pl
from jax.experimental.pallas import tpu as pltpu
from jax.experimental.pallas import tpu_sc as plsc

_NC = 2
_NS = 16
_NW = _NC * _NS


def _make_sc_gather(B, bpw):
  mesh = plsc.VectorSubcoreMesh(core_axis_name="c", subcore_axis_name="s")

  @functools.partial(
      pl.kernel,
      mesh=mesh,
      out_type=[
          jax.ShapeDtypeStruct((B * 4, 128), jnp.float32),
          jax.ShapeDtypeStruct((B * 4, 128), jnp.float32),
          jax.ShapeDtypeStruct((B * 4, 128), jnp.float32),
          jax.ShapeDtypeStruct((B * 4, 128), jnp.float32),
      ],
      scratch_types=[
          pltpu.VMEM((bpw,), jnp.int32),
          pltpu.VMEM((2, 128), jnp.int32),
          pltpu.VMEM((bpw * 8, 128), jnp.float32),
          pltpu.VMEM((bpw * 8, 128), jnp.float32),
          pltpu.SemaphoreType.DMA,
          pltpu.SemaphoreType.DMA,
          pltpu.SemaphoreType.DMA,
          pltpu.SemaphoreType.DMA,
          pltpu.SemaphoreType.DMA,
      ],
  )
  def gather_k(hid_hbm, cell_hbm, pos_hbm, h0out, h1out, c0out, c1out,
               pos_v, idx_v, hrows, crows, g0, g1, g2, g3, wsem):
    wid = lax.axis_index("s") * _NC + lax.axis_index("c")
    base = wid * bpw
    pltpu.sync_copy(pos_hbm.at[pl.ds(base, bpw)], pos_v)
    lane = lax.iota(jnp.int32, 16)
    dnum = lax.GatherDimensionNumbers(
        offset_dims=(), collapsed_slice_dims=(0,), start_index_map=(0,))
    for r in range(16):
      l = r // 8
      btl = (r // 2) & 3
      ht = 2 * (r % 2) + lax.shift_right_logical(lane, 3)
      bi = lax.bitwise_and(lane, 7)
      p16 = pos_v[pl.ds((btl // 2) * 16, 16)]
      gi = (btl & 1) * 8 + bi
      p = lax.gather(p16, gi[:, None], dnum, slice_sizes=(1,),
                     mode=lax.GatherScatterMode.PROMISE_IN_BOUNDS)
      b = base + btl * 8 + bi
      row = (p * B + b) * 8 + ht * 2 + l
      idx_v[r // 8, pl.ds((r % 8) * 16, 16)] = row
    gh0 = pltpu.async_copy(
        hid_hbm.at[idx_v.at[0]], hrows.at[pl.ds(0, 128)], g0)
    gc0 = pltpu.async_copy(
        cell_hbm.at[idx_v.at[0]], crows.at[pl.ds(0, 128)], g1)
    gh1 = pltpu.async_copy(
        hid_hbm.at[idx_v.at[1]], hrows.at[pl.ds(128, 128)], g2)
    gc1 = pltpu.async_copy(
        cell_hbm.at[idx_v.at[1]], crows.at[pl.ds(128, 128)], g3)
    dst = pl.ds(wid * 128, 128)
    gh0.wait()
    w0 = pltpu.async_copy(hrows.at[pl.ds(0, 128)], h0out.at[dst], wsem)
    gc0.wait()
    w2 = pltpu.async_copy(crows.at[pl.ds(0, 128)], c0out.at[dst], wsem)
    gh1.wait()
    w1 = pltpu.async_copy(hrows.at[pl.ds(128, 128)], h1out.at[dst], wsem)
    gc1.wait()
    w3 = pltpu.async_copy(crows.at[pl.ds(128, 128)], c1out.at[dst], wsem)
    for wcp in (w0, w2, w1, w3):
      wcp.wait()

  return gather_k


def _sig(x):
  return 0.5 * (jnp.tanh(0.5 * x) + 1.0)


def _lstm_body(x_ref, h0_ref, h1_ref, c0_ref, c1_ref,
               wi0_ref, wh0_ref, b0_ref, wi1_ref, wh1_ref, b1_ref, op_ref,
               nh3_ref, nc3_ref, top_ref, *, H):
  dn = (((1,), (1,)), ((), ()))
  prec = None
  h0, h1 = h0_ref[...], h1_ref[...]
  c0, c1 = c0_ref[...], c1_ref[...]
  x = x_ref[...]
  g0 = lax.dot_general(x, wi0_ref[...], dn, precision=prec,
                       preferred_element_type=jnp.float32)
  g0 = g0 + lax.dot_general(h0, wh0_ref[...], dn, precision=prec,
                            preferred_element_type=jnp.float32)
  g0 = g0 + b0_ref[...]
  i0, f0, gg0, o0 = (g0[:, :H], g0[:, H:2 * H], g0[:, 2 * H:3 * H],
                     g0[:, 3 * H:])
  cn0 = _sig(f0) * c0 + _sig(i0) * jnp.tanh(gg0)
  hn0 = _sig(o0) * jnp.tanh(cn0)
  g1 = lax.dot_general(hn0, wi1_ref[...], dn, precision=prec,
                       preferred_element_type=jnp.float32)
  g1 = g1 + lax.dot_general(h1, wh1_ref[...], dn, precision=prec,
                            preferred_element_type=jnp.float32)
  g1 = g1 + b1_ref[...]
  i1, f1, gg1, o1 = (g1[:, :H], g1[:, H:2 * H], g1[:, 2 * H:3 * H],
                     g1[:, 3 * H:])
  cn1 = _sig(f1) * c1 + _sig(i1) * jnp.tanh(gg1)
  hn1 = _sig(o1) * jnp.tanh(cn1)
  for ht in range(4):
    sl = slice(ht * 128, (ht + 1) * 128)
    nh3_ref[:, 2 * ht, :] = hn0[:, sl]
    nh3_ref[:, 2 * ht + 1, :] = hn1[:, sl]
    nc3_ref[:, 2 * ht, :] = cn0[:, sl]
    nc3_ref[:, 2 * ht + 1, :] = cn1[:, sl]
  top_ref[...] = jnp.where(op_ref[...] == 1, hn1, h1)


def _lstm_call(x, h0, h1, c0, c1, wi0, wh0, b0, wi1, wh1, b1, op2):
  B, IN = x.shape
  H = 512
  G = 4 * H
  bm = 256
  grid = (B // bm,)
  bspec_act = pl.BlockSpec((bm, IN), lambda i: (i, 0))
  bspec_h = pl.BlockSpec((bm, 512), lambda i: (i, 0))
  bspec_3 = pl.BlockSpec((bm, 8, 128), lambda i: (i, 0, 0))
  bspec_w = pl.BlockSpec((G, IN), lambda i: (0, 0))
  bspec_wh = pl.BlockSpec((G, H), lambda i: (0, 0))
  bspec_b = pl.BlockSpec((1, G), lambda i: (0, 0))
  bspec_op = pl.BlockSpec((bm, 1), lambda i: (i, 0))
  out_shape = [jax.ShapeDtypeStruct((B, 8, 128), jnp.float32),
               jax.ShapeDtypeStruct((B, 8, 128), jnp.float32),
               jax.ShapeDtypeStruct((B, H), jnp.float32)]
  out_specs = [bspec_3, bspec_3, bspec_h]
  return pl.pallas_call(
      functools.partial(_lstm_body, H=H),
      grid=grid,
      in_specs=[bspec_act, bspec_h, bspec_h, bspec_h, bspec_h,
                bspec_w, bspec_wh, bspec_b, bspec_wh, bspec_wh, bspec_b,
                bspec_op],
      out_specs=out_specs,
      out_shape=out_shape,
      compiler_params=pltpu.CompilerParams(
          dimension_semantics=("arbitrary",)),
  )(x, h0, h1, c0, c1, wi0, wh0, b0, wi1, wh1, b1, op2)


def kernel(input, hidden_stack, cell_stack, W_ih0, W_hh0, b0,
           W_ih1, W_hh1, b1, pos, op):
  B = input.shape[0]
  S1, _, H, L = hidden_stack.shape
  hid_t = (hidden_stack.reshape(S1, B, 4, 128, 2)
           .transpose(0, 1, 2, 4, 3).reshape(S1 * B * 8, 128))
  cell_t = (cell_stack.reshape(S1, B, 4, 128, 2)
            .transpose(0, 1, 2, 4, 3).reshape(S1 * B * 8, 128))
  pos32 = pos.astype(jnp.int32)
  bpw = B // _NW
  gather_k = _make_sc_gather(B, bpw)
  h0r, h1r, c0r, c1r = gather_k(hid_t, cell_t, pos32)

  def _as_bh(rows):
    return (rows.reshape(B // 8, 4, 8, 128).transpose(0, 2, 1, 3)
            .reshape(B, 512))

  h0, h1, c0, c1 = map(_as_bh, (h0r, h1r, c0r, c1r))
  op2 = op.astype(jnp.int32).reshape(B, 1)
  nh3, nc3, top = _lstm_call(
      input, h0, h1, c0, c1, W_ih0, W_hh0, b0.reshape(1, 4 * H),
      W_ih1, W_hh1, b1.reshape(1, 4 * H), op2)
  next_hidden = (nh3.reshape(B, 4, 2, 128).transpose(0, 1, 3, 2)
                 .reshape(B, H, L))
  next_cell = (nc3.reshape(B, 4, 2, 128).transpose(0, 1, 3, 2)
               .reshape(B, H, L))
  return (top, next_hidden, next_cell)
